# baseline (speedup 1.0000x reference)
import jax
import jax.numpy as jnp
from jax.experimental import pallas as pl
from jax.experimental.pallas import tpu as pltpu

B, N, D, O = 32, 512, 128, 128


def _fused_gat_kernel(x_ref, w_ref, a2_ref, bias_ref, out_ref):
    x = x_ref[0]
    W = w_ref[...]

    H = jnp.dot(x, W, preferred_element_type=jnp.float32)
    wa = jnp.dot(W, a2_ref[...], preferred_element_type=jnp.float32)
    sd = jnp.dot(x, wa, preferred_element_type=jnp.float32)
    s_col = sd[:, 0:1]
    d_col = sd[:, 1:2]
    s_row = sd[:, 0][None, :]

    max_s = jnp.max(s_col)
    zm = d_col + max_s
    m = jnp.maximum(zm, 0.2 * zm)

    u1 = jnp.exp(zm - m)
    u2 = jnp.exp(0.2 * zm - m)
    v1 = jnp.exp(s_row - max_s)
    v2 = jnp.exp(0.2 * (s_row - max_s))
    P = jnp.maximum(u1 * v1, u2 * v2)
    l = jnp.sum(P, axis=1, keepdims=True)

    out = jnp.dot(P, H, preferred_element_type=jnp.float32) * (1.0 / l)
    out = out + bias_ref[...]
    out_ref[0] = jnp.where(out > 0, out, jnp.exp(out) - 1.0)


def kernel(x, W, a_src, a_dst, bias):
    a2 = jnp.stack([a_src, a_dst], axis=1)
    bias2 = bias.reshape(1, O)

    return pl.pallas_call(
        _fused_gat_kernel,
        grid=(B,),
        in_specs=[
            pl.BlockSpec((1, N, D), lambda b: (b, 0, 0)),
            pl.BlockSpec((D, O), lambda b: (0, 0)),
            pl.BlockSpec((O, 2), lambda b: (0, 0)),
            pl.BlockSpec((1, O), lambda b: (0, 0)),
        ],
        out_specs=pl.BlockSpec((1, N, O), lambda b: (b, 0, 0)),
        out_shape=jax.ShapeDtypeStruct((B, N, O), jnp.float32),
        compiler_params=pltpu.CompilerParams(
            dimension_semantics=("parallel",),
        ),
    )(x, W, a2, bias2)


# speedup vs baseline: 1.6934x; 1.6934x over previous
import jax
import jax.numpy as jnp
from jax.experimental import pallas as pl
from jax.experimental.pallas import tpu as pltpu

B, N, D, O = 32, 512, 128, 128


BB = 4


def _fused_gat_kernel(x_ref, w_ref, a_src_ref, a_dst_ref, bias_ref, out_ref):
    W = w_ref[...]
    for i in range(BB):
        x = x_ref[i]
        H = jnp.dot(x, W, preferred_element_type=jnp.float32)

        s = jnp.sum(H * a_src_ref[...], axis=1)
        d = jnp.sum(H * a_dst_ref[...], axis=1)

        max_s = jnp.max(s)
        zm = d + max_s
        m = jnp.maximum(zm, 0.2 * zm)

        u1 = jnp.exp(zm - m)
        u2 = jnp.exp(0.2 * zm - m)
        v1 = jnp.exp(s - max_s)
        v2 = jnp.exp(0.2 * (s - max_s))
        P = jnp.maximum(u1[:, None] * v1[None, :],
                        u2[:, None] * v2[None, :])
        l = jnp.sum(P, axis=1, keepdims=True)

        out = jnp.dot(P, H, preferred_element_type=jnp.float32) * (1.0 / l)
        out = out + bias_ref[...]
        out_ref[i] = jnp.where(out > 0, out, jnp.exp(out) - 1.0)


def kernel(x, W, a_src, a_dst, bias):
    a_src2 = a_src.reshape(1, O)
    a_dst2 = a_dst.reshape(1, O)
    bias2 = bias.reshape(1, O)

    return pl.pallas_call(
        _fused_gat_kernel,
        grid=(B // BB,),
        in_specs=[
            pl.BlockSpec((BB, N, D), lambda b: (b, 0, 0)),
            pl.BlockSpec((D, O), lambda b: (0, 0)),
            pl.BlockSpec((1, O), lambda b: (0, 0)),
            pl.BlockSpec((1, O), lambda b: (0, 0)),
            pl.BlockSpec((1, O), lambda b: (0, 0)),
        ],
        out_specs=pl.BlockSpec((BB, N, O), lambda b: (b, 0, 0)),
        out_shape=jax.ShapeDtypeStruct((B, N, O), jnp.float32),
        compiler_params=pltpu.CompilerParams(
            dimension_semantics=("parallel",),
        ),
    )(x, W, a_src2, a_dst2, bias2)


# speedup vs baseline: 1.9719x; 1.1645x over previous
import jax
import jax.numpy as jnp
from jax.experimental import pallas as pl
from jax.experimental.pallas import tpu as pltpu

B, N, D, O = 32, 512, 128, 128


BB = 4


def _fused_gat_kernel(x_ref, w_ref, a_src_ref, a_dst_ref, bias_ref, out_ref):
    W = w_ref[...]
    for i in range(BB):
        x = x_ref[i]
        H = jnp.dot(x, W, preferred_element_type=jnp.float32)

        s = jnp.sum(H * a_src_ref[...], axis=1)
        d = jnp.sum(H * a_dst_ref[...], axis=1)

        E = d[:, None] + s[None, :]
        E = jnp.maximum(E, 0.2 * E)
        m = jnp.max(E, axis=1, keepdims=True)
        P = jnp.exp(E - m)
        l = jnp.sum(P, axis=1, keepdims=True)

        out = jnp.dot(P, H, preferred_element_type=jnp.float32) * (1.0 / l)
        out = out + bias_ref[...]
        out_ref[i] = jnp.where(out > 0, out, jnp.exp(out) - 1.0)


def kernel(x, W, a_src, a_dst, bias):
    a_src2 = a_src.reshape(1, O)
    a_dst2 = a_dst.reshape(1, O)
    bias2 = bias.reshape(1, O)

    return pl.pallas_call(
        _fused_gat_kernel,
        grid=(B // BB,),
        in_specs=[
            pl.BlockSpec((BB, N, D), lambda b: (b, 0, 0)),
            pl.BlockSpec((D, O), lambda b: (0, 0)),
            pl.BlockSpec((1, O), lambda b: (0, 0)),
            pl.BlockSpec((1, O), lambda b: (0, 0)),
            pl.BlockSpec((1, O), lambda b: (0, 0)),
        ],
        out_specs=pl.BlockSpec((BB, N, O), lambda b: (b, 0, 0)),
        out_shape=jax.ShapeDtypeStruct((B, N, O), jnp.float32),
        compiler_params=pltpu.CompilerParams(
            dimension_semantics=("parallel",),
        ),
    )(x, W, a_src2, a_dst2, bias2)
